# baseline (speedup 1.0000x reference)
import functools

import jax
import jax.numpy as jnp
from jax import lax
from jax.experimental import pallas as pl
from jax.experimental.pallas import tpu as pltpu
from jax.experimental.pallas import tpu_sc as plsc

B = 4096
T = 200
D = 64
NW = 32
ROWS_W = B // NW
HALF = T // 2
NBUF = 4
GROUPS = D // 16


def _make_sc_call():
  mesh = plsc.VectorSubcoreMesh(core_axis_name="c", subcore_axis_name="s")
  scratch = (
      [pltpu.VMEM((2 * ROWS_W, HALF), jnp.int32)]
      + [pltpu.VMEM((T, D), jnp.float32)]
      + [pltpu.VMEM((T, D), jnp.float32)] * NBUF
      + [pltpu.SemaphoreType.DMA] * NBUF
      + [pltpu.SemaphoreType.DMA] * NBUF
  )

  @functools.partial(
      pl.kernel,
      out_type=jax.ShapeDtypeStruct((B, T, D), jnp.float32),
      mesh=mesh,
      scratch_types=scratch,
      compiler_params=pltpu.CompilerParams(use_tc_tiling_on_sc=False),
  )
  def sc_embed(ids_hbm, tok_hbm, pos_hbm, out_hbm, idx_v, pos_v, *rest):
    bufs = rest[:NBUF]
    gsem = rest[NBUF:2 * NBUF]
    osem = rest[2 * NBUF:]

    wid = lax.axis_index("s") * 2 + lax.axis_index("c")
    row_base = wid * ROWS_W

    pltpu.sync_copy(ids_hbm.at[wid], idx_v)
    pltpu.sync_copy(pos_hbm.at[pl.ds(0, T)], pos_v)

    def gather_start(r, b):
      pltpu.async_copy(
          tok_hbm.at[idx_v.at[2 * r]], bufs[b].at[pl.ds(0, HALF)], gsem[b])
      pltpu.async_copy(
          tok_hbm.at[idx_v.at[2 * r + 1]], bufs[b].at[pl.ds(HALF, HALF)],
          gsem[b])

    def gather_wait(r, b):
      pltpu.make_async_copy(
          tok_hbm.at[idx_v.at[2 * r]], bufs[b].at[pl.ds(0, HALF)],
          gsem[b]).wait()
      pltpu.make_async_copy(
          tok_hbm.at[idx_v.at[2 * r + 1]], bufs[b].at[pl.ds(HALF, HALF)],
          gsem[b]).wait()

    def out_ref(r):
      return out_hbm.at[row_base + r]

    def add_pos(b):
      @plsc.parallel_loop(0, T, unroll=8)
      def _(t):
        for g in range(GROUPS):
          sl = pl.ds(g * 16, 16)
          plsc.addupdate(bufs[b].at[t, sl], pos_v[t, sl])

    def process(r, b):
      gather_wait(r, b)
      add_pos(b)
      pltpu.async_copy(bufs[b], out_ref(r), osem[b])

    def refill(r, r_next, b):
      pltpu.make_async_copy(bufs[b], out_ref(r), osem[b]).wait()
      gather_start(r_next, b)

    for b in range(NBUF):
      gather_start(b, b)

    n_rounds = ROWS_W // NBUF

    def round_body(g, carry):
      for b in range(NBUF):
        process(g * NBUF + b, b)
      for b in range(NBUF):
        refill(g * NBUF + b, (g + 1) * NBUF + b, b)
      return carry

    lax.fori_loop(0, n_rounds - 1, round_body, 0, unroll=False)

    g = n_rounds - 1
    for b in range(NBUF):
      process(g * NBUF + b, b)
    for b in range(NBUF):
      pltpu.make_async_copy(bufs[b], out_ref(g * NBUF + b), osem[b]).wait()

  return sc_embed


_SC_EMBED = _make_sc_call()


@jax.jit
def kernel(x_ids, tok_weight, pos_weight):
  ids = x_ids.astype(jnp.int32).reshape(NW, 2 * ROWS_W, HALF)
  return _SC_EMBED(ids, tok_weight, pos_weight)


# speedup vs baseline: 1.0649x; 1.0649x over previous
import functools

import jax
import jax.numpy as jnp
from jax import lax
from jax.experimental import pallas as pl
from jax.experimental.pallas import tpu as pltpu
from jax.experimental.pallas import tpu_sc as plsc

B = 4096
T = 200
D = 64
PAD_D = 128
NW = 32
BW = B // NW
NB_G = 4
NB_C = 2
GROUPS = D // 16


def _make_sc_call():
  mesh = plsc.VectorSubcoreMesh(core_axis_name="c", subcore_axis_name="s")
  scratch = (
      [pltpu.VMEM((T, BW), jnp.int32)]
      + [pltpu.VMEM((T * D,), jnp.float32)]
      + [pltpu.VMEM((BW, PAD_D), jnp.float32)] * NB_G
      + [pltpu.VMEM((D, BW), jnp.float32)] * NB_C
      + [pltpu.SemaphoreType.DMA] * NB_G
      + [pltpu.SemaphoreType.DMA] * NB_C
  )

  @functools.partial(
      pl.kernel,
      out_type=jax.ShapeDtypeStruct((T, D, B), jnp.float32),
      mesh=mesh,
      scratch_types=scratch,
      compiler_params=pltpu.CompilerParams(needs_layout_passes=False),
  )
  def sc_embed(ids_hbm, tok_hbm, pos_hbm, out_hbm, idx_v, pos_v, *rest):
    gbuf = rest[:NB_G]
    cbuf = rest[NB_G:NB_G + NB_C]
    gsem = rest[NB_G + NB_C:2 * NB_G + NB_C]
    osem = rest[2 * NB_G + NB_C:]

    wid = lax.axis_index("s") * 2 + lax.axis_index("c")
    b0 = wid * BW

    pltpu.sync_copy(ids_hbm.at[:, pl.ds(b0, BW)], idx_v)
    pltpu.sync_copy(pos_hbm, pos_v)

    iota16 = lax.iota(jnp.int32, 16)
    jdx = [iota16 + 16 * k for k in range(BW // 16)]

    def gather_start(t, b):
      pltpu.async_copy(tok_hbm.at[idx_v.at[t]], gbuf[b], gsem[b])

    def gather_wait(t, b):
      pltpu.make_async_copy(tok_hbm.at[idx_v.at[t]], gbuf[b], gsem[b]).wait()

    def out_ref(t):
      return out_hbm.at[t, :, pl.ds(b0, BW)]

    def build(t, b, c):
      @plsc.parallel_loop(0, D, unroll=8)
      def _(d):
        pv = plsc.load_gather(
            pos_v, [jnp.full((16,), t * D + d, dtype=jnp.int32)])
        colv = jnp.full((16,), d, dtype=jnp.int32)
        for k in range(BW // 16):
          v = plsc.load_gather(gbuf[b], [jdx[k], colv]) + pv
          cbuf[c][d, pl.ds(16 * k, 16)] = v

    def store_wait(t, c):
      pltpu.make_async_copy(cbuf[c], out_ref(t), osem[c]).wait()

    def round_body(g, first=False, refill=True):
      t0 = g * NB_G
      for b in range(NB_G):
        c = b % NB_C
        t = t0 + b
        if not (first and b < NB_C):
          store_wait(t - NB_C, c)
        gather_wait(t, b)
        build(t, b, c)
        if refill:
          gather_start(t + NB_G, b)
        pltpu.async_copy(cbuf[c], out_ref(t), osem[c])

    for b in range(NB_G):
      gather_start(b, b)

    n_rounds = T // NB_G

    round_body(0, first=True, refill=True)

    def fori_body(g, carry):
      round_body(g, first=False, refill=True)
      return carry

    lax.fori_loop(1, n_rounds - 1, fori_body, 0, unroll=False)

    round_body(n_rounds - 1, first=False, refill=False)

    store_wait(T - 2, 0)
    store_wait(T - 1, 1)

  return sc_embed


_SC_EMBED = _make_sc_call()


@jax.jit
def kernel(x_ids, tok_weight, pos_weight):
  ids_tb = jnp.transpose(x_ids.astype(jnp.int32), (1, 0))
  tok_padded = jnp.pad(tok_weight, ((0, 0), (0, PAD_D - D)))
  pos_flat = pos_weight[:T].reshape(-1)
  out_tdb = _SC_EMBED(ids_tb, tok_padded, pos_flat)
  return jnp.transpose(out_tdb, (2, 0, 1))


# speedup vs baseline: 1.6887x; 1.5857x over previous
import functools

import jax
import jax.numpy as jnp
from jax import lax
from jax.experimental import pallas as pl
from jax.experimental.pallas import tpu as pltpu
from jax.experimental.pallas import tpu_sc as plsc

B = 4096
T = 200
D = 64
PAD_D = 128
NW = 32
BW = B // NW
NB_G = 4
NB_C = 2
GROUPS = D // 16


def _make_sc_call():
  mesh = plsc.VectorSubcoreMesh(core_axis_name="c", subcore_axis_name="s")
  scratch = (
      [pltpu.VMEM((T, BW), jnp.int32)]
      + [pltpu.VMEM((T * D,), jnp.float32)]
      + [pltpu.VMEM((BW, PAD_D), jnp.float32)] * NB_G
      + [pltpu.VMEM((D, BW), jnp.float32)] * NB_C
      + [pltpu.SemaphoreType.DMA] * NB_G
      + [pltpu.SemaphoreType.DMA] * NB_C
  )

  @functools.partial(
      pl.kernel,
      out_type=jax.ShapeDtypeStruct((T, D, B), jnp.float32),
      mesh=mesh,
      scratch_types=scratch,
      compiler_params=pltpu.CompilerParams(needs_layout_passes=False),
  )
  def sc_embed(ids_hbm, tok_hbm, pos_hbm, out_hbm, idx_v, pos_v, *rest):
    gbuf = rest[:NB_G]
    cbuf = rest[NB_G:NB_G + NB_C]
    gsem = rest[NB_G + NB_C:2 * NB_G + NB_C]
    osem = rest[2 * NB_G + NB_C:]

    wid = lax.axis_index("s") * 2 + lax.axis_index("c")
    b0 = wid * BW

    pltpu.sync_copy(ids_hbm.at[:, pl.ds(b0, BW)], idx_v)
    pltpu.sync_copy(pos_hbm, pos_v)

    iota16 = lax.iota(jnp.int32, 16)
    jdx = [iota16 + 16 * k for k in range(BW // 16)]

    def gather_start(t, b):
      pltpu.async_copy(tok_hbm.at[idx_v.at[t]], gbuf[b], gsem[b])

    def gather_wait(t, b):
      pltpu.make_async_copy(tok_hbm.at[idx_v.at[t]], gbuf[b], gsem[b]).wait()

    def out_ref(t):
      return out_hbm.at[t, :, pl.ds(b0, BW)]

    def build(t, b, c):
      @plsc.parallel_loop(0, D, unroll=4)
      def _(d):
        dsplat = jnp.full((16,), d, dtype=jnp.int32)
        colv = ((dsplat + iota16) & 15) | (dsplat & ~15)
        pv = plsc.load_gather(pos_v, [colv + t * D])
        for k in range(BW // 16):
          v = plsc.load_gather(gbuf[b], [jdx[k], colv]) + pv
          plsc.store_scatter(cbuf[c], [colv, jdx[k]], v)

    def store_wait(t, c):
      pltpu.make_async_copy(cbuf[c], out_ref(t), osem[c]).wait()

    def round_body(g, first=False, refill=True):
      t0 = g * NB_G
      for b in range(NB_G):
        c = b % NB_C
        t = t0 + b
        if not (first and b < NB_C):
          store_wait(t - NB_C, c)
        gather_wait(t, b)
        build(t, b, c)
        if refill:
          gather_start(t + NB_G, b)
        pltpu.async_copy(cbuf[c], out_ref(t), osem[c])

    for b in range(NB_G):
      gather_start(b, b)

    n_rounds = T // NB_G

    round_body(0, first=True, refill=True)

    def fori_body(g, carry):
      round_body(g, first=False, refill=True)
      return carry

    lax.fori_loop(1, n_rounds - 1, fori_body, 0, unroll=False)

    round_body(n_rounds - 1, first=False, refill=False)

    store_wait(T - 2, 0)
    store_wait(T - 1, 1)

  return sc_embed


_SC_EMBED = _make_sc_call()


@jax.jit
def kernel(x_ids, tok_weight, pos_weight):
  ids_tb = jnp.transpose(x_ids.astype(jnp.int32), (1, 0))
  tok_padded = jnp.pad(tok_weight, ((0, 0), (0, PAD_D - D)))
  pos_flat = pos_weight[:T].reshape(-1)
  out_tdb = _SC_EMBED(ids_tb, tok_padded, pos_flat)
  return jnp.transpose(out_tdb, (2, 0, 1))


# speedup vs baseline: 2.4148x; 1.4300x over previous
import functools

import jax
import jax.numpy as jnp
from jax import lax
from jax.experimental import pallas as pl
from jax.experimental.pallas import tpu as pltpu
from jax.experimental.pallas import tpu_sc as plsc

B = 4096
T = 200
D = 64
PAD_D = 128
NW = 32
BW = B // NW
NB_G = 4
NB_C = 2
GROUPS = D // 16


def _make_sc_call():
  mesh = plsc.VectorSubcoreMesh(core_axis_name="c", subcore_axis_name="s")
  scratch = (
      [pltpu.VMEM((T, BW), jnp.int32)]
      + [pltpu.VMEM((T * D,), jnp.float32)]
      + [pltpu.VMEM((BW, PAD_D), jnp.float32)] * NB_G
      + [pltpu.VMEM((D, BW), jnp.float32)] * NB_C
      + [pltpu.SemaphoreType.DMA] * NB_G
      + [pltpu.SemaphoreType.DMA] * NB_C
  )

  @functools.partial(
      pl.kernel,
      out_type=jax.ShapeDtypeStruct((T, D, B), jnp.float32),
      mesh=mesh,
      scratch_types=scratch,
      compiler_params=pltpu.CompilerParams(needs_layout_passes=False),
  )
  def sc_embed(ids_hbm, tok_hbm, pos_hbm, out_hbm, idx_v, pos_v, *rest):
    gbuf = rest[:NB_G]
    cbuf = rest[NB_G:NB_G + NB_C]
    gsem = rest[NB_G + NB_C:2 * NB_G + NB_C]
    osem = rest[2 * NB_G + NB_C:]

    wid = lax.axis_index("s") * 2 + lax.axis_index("c")
    b0 = wid * BW

    pltpu.sync_copy(ids_hbm.at[:, pl.ds(b0, BW)], idx_v)
    pltpu.sync_copy(pos_hbm, pos_v)

    iota16 = lax.iota(jnp.int32, 16)
    jdx = [iota16 + 16 * k for k in range(BW // 16)]

    def gather_start(t, b):
      pltpu.async_copy(tok_hbm.at[idx_v.at[t]], gbuf[b], gsem[b])

    def gather_wait(t, b):
      pltpu.make_async_copy(tok_hbm.at[idx_v.at[t]], gbuf[b], gsem[b]).wait()

    def out_ref(t):
      return out_hbm.at[t, :, pl.ds(b0, BW)]

    def build(t, b, c):
      @plsc.parallel_loop(0, D, unroll=4)
      def _(d):
        dsplat = jnp.full((16,), d, dtype=jnp.int32)
        colv = ((dsplat + iota16) & 15) | (dsplat & ~15)
        pv = plsc.load_gather(pos_v, [colv + t * D])
        for k in range(BW // 16):
          v = plsc.load_gather(gbuf[b], [jdx[k], colv]) + pv
          plsc.store_scatter(cbuf[c], [colv, jdx[k]], v)

    def store_wait(t, c):
      pltpu.make_async_copy(cbuf[c], out_ref(t), osem[c]).wait()

    def round_body(g, first=False, refill=True):
      t0 = g * NB_G
      for b in range(NB_G):
        c = b % NB_C
        t = t0 + b
        if not (first and b < NB_C):
          store_wait(t - NB_C, c)
        gather_wait(t, b)
        build(t, b, c)
        if refill:
          gather_start(t + NB_G, b)
        pltpu.async_copy(cbuf[c], out_ref(t), osem[c])

    for b in range(NB_G):
      gather_start(b, b)

    n_rounds = T // NB_G

    round_body(0, first=True, refill=True)

    def fori_body(g, carry):
      round_body(g, first=False, refill=True)
      return carry

    lax.fori_loop(1, n_rounds - 1, fori_body, 0, unroll=False)

    round_body(n_rounds - 1, first=False, refill=False)

    store_wait(T - 2, 0)
    store_wait(T - 1, 1)

  return sc_embed


VOCAB = 1000000
NWIN = VOCAB // 128
NTAIL = VOCAB - NWIN * 128
WFULL = NWIN // NW
VBUF_N = 2


def _make_tr_call():
  mesh = plsc.VectorSubcoreMesh(core_axis_name="c", subcore_axis_name="s")
  scratch = (
      [pltpu.VMEM((D, 128), jnp.float32)] * VBUF_N
      + [pltpu.VMEM((128, 128), jnp.float32)] * VBUF_N
      + [pltpu.VMEM((NTAIL, D), jnp.float32)]
      + [pltpu.SemaphoreType.DMA] * VBUF_N
      + [pltpu.SemaphoreType.DMA] * VBUF_N
  )

  @functools.partial(
      pl.kernel,
      out_type=jax.ShapeDtypeStruct((VOCAB, PAD_D), jnp.float32),
      mesh=mesh,
      scratch_types=scratch,
      compiler_params=pltpu.CompilerParams(needs_layout_passes=False),
  )
  def sc_tr(src_hbm, tail_hbm, dst_hbm, *rest):
    vbuf = rest[:VBUF_N]
    tbuf = rest[VBUF_N:2 * VBUF_N]
    tailbuf = rest[2 * VBUF_N]
    ssem = rest[2 * VBUF_N + 1:3 * VBUF_N + 1]
    osem = rest[3 * VBUF_N + 1:]

    wid = lax.axis_index("s") * 2 + lax.axis_index("c")

    iota16 = lax.iota(jnp.int32, 16)
    jdx = [iota16 + 16 * k for k in range(8)]

    def win_v0(i):
      return pl.multiple_of((i * NW + wid) * 128, 128)

    def stage_start(v0, b):
      pltpu.async_copy(src_hbm.at[:, pl.ds(v0, 128)], vbuf[b], ssem[b])

    def stage_wait(v0, b):
      pltpu.make_async_copy(
          src_hbm.at[:, pl.ds(v0, 128)], vbuf[b], ssem[b]).wait()

    def store_start(v0, b):
      pltpu.async_copy(tbuf[b], dst_hbm.at[pl.ds(v0, 128)], osem[b])

    def store_wait(v0, b):
      pltpu.make_async_copy(
          tbuf[b], dst_hbm.at[pl.ds(v0, 128)], osem[b]).wait()

    def transpose_into(src_ref, dst_ref, n_v):
      @plsc.parallel_loop(0, n_v, unroll=2)
      def _(x):
        s = x & 15
        kv = lax.shift_right_logical(x, 4)
        dvec = (iota16 + s) & 15
        vvec = iota16 + 16 * kv
        for kd in range(GROUPS):
          dv = dvec + 16 * kd
          val = plsc.load_gather(src_ref, [dv, vvec])
          plsc.store_scatter(dst_ref, [vvec, dv], val)

    def transpose(b):
      transpose_into(vbuf[b], tbuf[b], 128)

    for b in range(VBUF_N):
      @pl.when(b * NW + wid < NWIN)
      def _(b=b):
        stage_start(win_v0(b), b)

    def round_body(g, carry):
      for b in range(VBUF_N):
        ii = g * VBUF_N + b

        @pl.when(ii * NW + wid < NWIN)
        def _(ii=ii, b=b):
          v0 = win_v0(ii)
          stage_wait(v0, b)

          @pl.when(ii >= VBUF_N)
          def _():
            store_wait(win_v0(ii - VBUF_N), b)

          transpose(b)
          store_start(v0, b)
          nxt = ii + VBUF_N

          @pl.when(nxt * NW + wid < NWIN)
          def _():
            stage_start(win_v0(nxt), b)

      return carry

    n_rounds = (WFULL + 1 + VBUF_N - 1) // VBUF_N + 1
    lax.fori_loop(0, n_rounds, round_body, 0, unroll=False)

    last0 = jnp.where(wid < NWIN - NW * WFULL, WFULL, WFULL - 2)
    store_wait(win_v0(last0), 0)
    store_wait(win_v0(WFULL - 1), 1)

    @pl.when(wid == 0)
    def _():
      pltpu.sync_copy(tail_hbm, tailbuf)

      @plsc.parallel_loop(0, NTAIL, unroll=2)
      def _(v):
        for k in range(GROUPS):
          tbuf[0][v, pl.ds(16 * k, 16)] = tailbuf[v, pl.ds(16 * k, 16)]

      pltpu.sync_copy(tbuf[0].at[pl.ds(0, NTAIL)],
                      dst_hbm.at[pl.ds(NWIN * 128, NTAIL)])

  return sc_tr


_SC_EMBED = _make_sc_call()
_SC_TR = _make_tr_call()


@jax.jit
def kernel(x_ids, tok_weight, pos_weight):
  ids_tb = jnp.transpose(x_ids.astype(jnp.int32), (1, 0))
  tok_padded = _SC_TR(jnp.transpose(tok_weight, (1, 0)),
                      lax.slice(tok_weight, (NWIN * 128, 0), (VOCAB, D)))
  pos_flat = pos_weight[:T].reshape(-1)
  out_tdb = _SC_EMBED(ids_tb, tok_padded, pos_flat)
  return jnp.transpose(out_tdb, (2, 0, 1))


# speedup vs baseline: 2.4308x; 1.0066x over previous
import functools

import jax
import jax.numpy as jnp
from jax import lax
from jax.experimental import pallas as pl
from jax.experimental.pallas import tpu as pltpu
from jax.experimental.pallas import tpu_sc as plsc

B = 4096
T = 200
D = 64
PAD_D = 128
NW = 32
BW = B // NW
NB_G = 4
NB_C = 2
GROUPS = D // 16


def _make_sc_call():
  mesh = plsc.VectorSubcoreMesh(core_axis_name="c", subcore_axis_name="s")
  scratch = (
      [pltpu.VMEM((T, BW), jnp.int32)]
      + [pltpu.VMEM((T * D,), jnp.float32)]
      + [pltpu.VMEM((BW, PAD_D), jnp.float32)] * NB_G
      + [pltpu.VMEM((D, BW), jnp.float32)] * NB_C
      + [pltpu.SemaphoreType.DMA] * NB_G
      + [pltpu.SemaphoreType.DMA] * NB_C
  )

  @functools.partial(
      pl.kernel,
      out_type=jax.ShapeDtypeStruct((T, D, B), jnp.float32),
      mesh=mesh,
      scratch_types=scratch,
      compiler_params=pltpu.CompilerParams(needs_layout_passes=False),
  )
  def sc_embed(ids_hbm, tok_hbm, pos_hbm, out_hbm, idx_v, pos_v, *rest):
    gbuf = rest[:NB_G]
    cbuf = rest[NB_G:NB_G + NB_C]
    gsem = rest[NB_G + NB_C:2 * NB_G + NB_C]
    osem = rest[2 * NB_G + NB_C:]

    wid = lax.axis_index("s") * 2 + lax.axis_index("c")
    b0 = wid * BW

    pltpu.sync_copy(ids_hbm.at[:, pl.ds(b0, BW)], idx_v)
    pltpu.sync_copy(pos_hbm, pos_v)

    iota16 = lax.iota(jnp.int32, 16)
    jdx = [iota16 + 16 * k for k in range(BW // 16)]

    def gather_start(t, b):
      pltpu.async_copy(tok_hbm.at[idx_v.at[t]], gbuf[b], gsem[b])

    def gather_wait(t, b):
      pltpu.make_async_copy(tok_hbm.at[idx_v.at[t]], gbuf[b], gsem[b]).wait()

    def out_ref(t):
      return out_hbm.at[t, :, pl.ds(b0, BW)]

    def build(t, b, c):
      @plsc.parallel_loop(0, D, unroll=4)
      def _(d):
        dsplat = jnp.full((16,), d, dtype=jnp.int32)
        colv = ((dsplat + iota16) & 15) | (dsplat & ~15)
        pv = plsc.load_gather(pos_v, [colv + t * D])
        for k in range(BW // 16):
          v = plsc.load_gather(gbuf[b], [jdx[k], colv]) + pv
          plsc.store_scatter(cbuf[c], [colv, jdx[k]], v)

    def store_wait(t, c):
      pltpu.make_async_copy(cbuf[c], out_ref(t), osem[c]).wait()

    def round_body(g, first=False, refill=True):
      t0 = g * NB_G
      for b in range(NB_G):
        c = b % NB_C
        t = t0 + b
        if not (first and b < NB_C):
          store_wait(t - NB_C, c)
        gather_wait(t, b)
        build(t, b, c)
        if refill:
          gather_start(t + NB_G, b)
        pltpu.async_copy(cbuf[c], out_ref(t), osem[c])

    for b in range(NB_G):
      gather_start(b, b)

    n_rounds = T // NB_G

    round_body(0, first=True, refill=True)

    def fori_body(g, carry):
      round_body(g, first=False, refill=True)
      return carry

    lax.fori_loop(1, n_rounds - 1, fori_body, 0, unroll=False)

    round_body(n_rounds - 1, first=False, refill=False)

    store_wait(T - 2, 0)
    store_wait(T - 1, 1)

  return sc_embed


VOCAB = 1000000
NWIN = VOCAB // 128
NTAIL = VOCAB - NWIN * 128
WFULL = NWIN // NW
VBUF_N = 3


def _make_tr_call():
  mesh = plsc.VectorSubcoreMesh(core_axis_name="c", subcore_axis_name="s")
  scratch = (
      [pltpu.VMEM((D, 128), jnp.float32)] * VBUF_N
      + [pltpu.VMEM((128, 128), jnp.float32)] * VBUF_N
      + [pltpu.VMEM((NTAIL, D), jnp.float32)]
      + [pltpu.SemaphoreType.DMA] * VBUF_N
      + [pltpu.SemaphoreType.DMA] * VBUF_N
  )

  @functools.partial(
      pl.kernel,
      out_type=jax.ShapeDtypeStruct((VOCAB, PAD_D), jnp.float32),
      mesh=mesh,
      scratch_types=scratch,
      compiler_params=pltpu.CompilerParams(needs_layout_passes=False),
  )
  def sc_tr(src_hbm, tail_hbm, dst_hbm, *rest):
    vbuf = rest[:VBUF_N]
    tbuf = rest[VBUF_N:2 * VBUF_N]
    tailbuf = rest[2 * VBUF_N]
    ssem = rest[2 * VBUF_N + 1:3 * VBUF_N + 1]
    osem = rest[3 * VBUF_N + 1:]

    wid = lax.axis_index("s") * 2 + lax.axis_index("c")

    iota16 = lax.iota(jnp.int32, 16)
    jdx = [iota16 + 16 * k for k in range(8)]

    def win_v0(i):
      return pl.multiple_of((i * NW + wid) * 128, 128)

    def stage_start(v0, b):
      pltpu.async_copy(src_hbm.at[:, pl.ds(v0, 128)], vbuf[b], ssem[b])

    def stage_wait(v0, b):
      pltpu.make_async_copy(
          src_hbm.at[:, pl.ds(v0, 128)], vbuf[b], ssem[b]).wait()

    def store_start(v0, b):
      pltpu.async_copy(tbuf[b], dst_hbm.at[pl.ds(v0, 128)], osem[b])

    def store_wait(v0, b):
      pltpu.make_async_copy(
          tbuf[b], dst_hbm.at[pl.ds(v0, 128)], osem[b]).wait()

    def transpose_into(src_ref, dst_ref, n_v):
      @plsc.parallel_loop(0, n_v, unroll=4)
      def _(x):
        s = x & 15
        kv = lax.shift_right_logical(x, 4)
        dvec = (iota16 + s) & 15
        vvec = iota16 + 16 * kv
        for kd in range(GROUPS):
          dv = dvec + 16 * kd
          val = plsc.load_gather(src_ref, [dv, vvec])
          plsc.store_scatter(dst_ref, [vvec, dv], val)

    def transpose(b):
      transpose_into(vbuf[b], tbuf[b], 128)

    for b in range(VBUF_N):
      @pl.when(b * NW + wid < NWIN)
      def _(b=b):
        stage_start(win_v0(b), b)

    def round_body(g, carry):
      for b in range(VBUF_N):
        ii = g * VBUF_N + b

        @pl.when(ii * NW + wid < NWIN)
        def _(ii=ii, b=b):
          v0 = win_v0(ii)
          stage_wait(v0, b)

          @pl.when(ii >= VBUF_N)
          def _():
            store_wait(win_v0(ii - VBUF_N), b)

          transpose(b)
          store_start(v0, b)
          nxt = ii + VBUF_N

          @pl.when(nxt * NW + wid < NWIN)
          def _():
            stage_start(win_v0(nxt), b)

      return carry

    n_rounds = (WFULL + 1 + VBUF_N - 1) // VBUF_N + 1
    lax.fori_loop(0, n_rounds, round_body, 0, unroll=False)

    last0 = jnp.where(wid < NWIN - NW * WFULL, WFULL, WFULL - 2)
    store_wait(win_v0(last0), 0)
    store_wait(win_v0(WFULL - 1), 1)

    @pl.when(wid == 0)
    def _():
      pltpu.sync_copy(tail_hbm, tailbuf)

      @plsc.parallel_loop(0, NTAIL, unroll=2)
      def _(v):
        for k in range(GROUPS):
          tbuf[0][v, pl.ds(16 * k, 16)] = tailbuf[v, pl.ds(16 * k, 16)]

      pltpu.sync_copy(tbuf[0].at[pl.ds(0, NTAIL)],
                      dst_hbm.at[pl.ds(NWIN * 128, NTAIL)])

  return sc_tr


_SC_EMBED = _make_sc_call()
_SC_TR = _make_tr_call()


@jax.jit
def kernel(x_ids, tok_weight, pos_weight):
  ids_tb = jnp.transpose(x_ids.astype(jnp.int32), (1, 0))
  tok_padded = _SC_TR(jnp.transpose(tok_weight, (1, 0)),
                      lax.slice(tok_weight, (NWIN * 128, 0), (VOCAB, D)))
  pos_flat = pos_weight[:T].reshape(-1)
  out_tdb = _SC_EMBED(ids_tb, tok_padded, pos_flat)
  return jnp.transpose(out_tdb, (2, 0, 1))
